# Initial kernel scaffold; baseline (speedup 1.0000x reference)
import jax
import jax.numpy as jnp
from jax.experimental import pallas as pl


def kernel(input_seq, token_table, pos_table):
    raise NotImplementedError("write your pallas kernel here")


# speedup vs baseline: 2.5461x; 2.5461x over previous
import jax
import jax.numpy as jnp
from jax import lax
from jax.experimental import pallas as pl
from jax.experimental.pallas import tpu as pltpu
from jax.experimental.pallas import tpu_sc as plsc

VOCAB = 100000
EMBED = 64
MAX_LEN = 200
BATCH = 4096

G = 100
NGROUPS = BATCH * MAX_LEN // G
NW = 32
GROUPS_PER_W = NGROUPS // NW
K = 4
CHUNKS = GROUPS_PER_W // K


def _body(in_ref, tok_ref, pos_ref, out_ref, idx_v, rows_v, sem):
    cid = lax.axis_index("c")
    sid = lax.axis_index("s")
    wid = sid * 2 + cid
    base_group = wid * GROUPS_PER_W

    @pl.loop(0, CHUNKS)
    def _chunk(c):
        row = base_group + c * K
        pre = [pltpu.async_copy(in_ref.at[pl.ds(row, K)], idx_v, sem)]
        for j in range(K):
            pre.append(pltpu.async_copy(pos_ref.at[j % 2], rows_v.at[j], sem))
        for d in pre:
            d.wait()
        descs = []
        for j in range(K):
            descs.append(
                pltpu.async_copy(tok_ref.at[idx_v.at[j]], rows_v.at[j], sem,
                                 add=True))
        for d in descs:
            d.wait()
        pltpu.sync_copy(rows_v, out_ref.at[pl.ds(row, K)])


def kernel(input_seq, token_table, pos_table):
    idx2d = input_seq.astype(jnp.int32).reshape(NGROUPS, G)
    pos3d = pos_table.reshape(2, G, EMBED)

    mesh = plsc.VectorSubcoreMesh(core_axis_name="c", subcore_axis_name="s")
    run = pl.kernel(
        _body,
        out_type=jax.ShapeDtypeStruct((NGROUPS, G, EMBED), jnp.float32),
        mesh=mesh,
        scratch_types=[
            pltpu.VMEM((K, G), jnp.int32),
            pltpu.VMEM((K, G, EMBED), jnp.float32),
            pltpu.SemaphoreType.DMA,
        ],
        compiler_params=pltpu.CompilerParams(use_tc_tiling_on_sc=False),
    )
    out = run(idx2d, token_table, pos3d)
    return out.reshape(BATCH, MAX_LEN, EMBED)
